# Initial kernel scaffold; baseline (speedup 1.0000x reference)
import jax
import jax.numpy as jnp
from jax.experimental import pallas as pl


def kernel(x, edge_index, W0, b0, W1, b1):
    raise NotImplementedError("write your pallas kernel here")


# speedup vs baseline: 7.4143x; 7.4143x over previous
import functools

import jax
import jax.numpy as jnp
from jax import lax
from jax.experimental import pallas as pl
from jax.experimental.pallas import tpu as pltpu
from jax.experimental.pallas import tpu_sc as plsc

N = 10000
E = 320000
D = 128
NP = 10240
NC = 2
NS = 16
NW = NC * NS
EPW = E // NW
CH = 80
NCHUNK = EPW // CH
RPT = NP // NS
BR = 640

_f32 = jnp.float32
_mesh = plsc.VectorSubcoreMesh(
    core_axis_name="c", subcore_axis_name="s", num_cores=NC, num_subcores=NS
)


def _deg_body(src_hbm, dst_hbm, od_out, id_out,
              src_v, dst_v, ones_v, z16_v, cb_v, sp_od, sp_id):
    c = lax.axis_index("c")
    s = lax.axis_index("s")
    wid = s * NC + c

    z16_v[...] = jnp.zeros((16,), _f32)
    for i in range(CH // 16):
        ones_v[pl.ds(16 * i, 16)] = jnp.ones((16,), _f32)

    def _z(k, _):
        pltpu.sync_copy(z16_v, sp_od.at[pl.ds(s * RPT + 16 * k, 16)])
        pltpu.sync_copy(z16_v, sp_id.at[pl.ds(s * RPT + 16 * k, 16)])
        return _
    lax.fori_loop(0, RPT // 16, _z, None)

    pltpu.sync_copy(src_hbm.at[wid], src_v)
    pltpu.sync_copy(dst_hbm.at[wid], dst_v)
    plsc.subcore_barrier()

    def _acc(j, _):
        pltpu.sync_copy(ones_v, sp_od.at[src_v.at[j]], add=True)
        pltpu.sync_copy(ones_v, sp_id.at[dst_v.at[j]], add=True)
        return _
    lax.fori_loop(0, NCHUNK, _acc, None)
    plsc.subcore_barrier()

    pltpu.sync_copy(sp_od.at[pl.ds(s * RPT, RPT)], cb_v)
    pltpu.sync_copy(cb_v, od_out.at[c, pl.ds(s * RPT, RPT)])
    pltpu.sync_copy(sp_id.at[pl.ds(s * RPT, RPT)], cb_v)
    pltpu.sync_copy(cb_v, id_out.at[c, pl.ds(s * RPT, RPT)])


_sc_deg = pl.kernel(
    _deg_body,
    out_type=[jax.ShapeDtypeStruct((NC, NP), _f32),
              jax.ShapeDtypeStruct((NC, NP), _f32)],
    mesh=_mesh,
    scratch_types=[
        pltpu.VMEM((NCHUNK, CH), jnp.int32),
        pltpu.VMEM((NCHUNK, CH), jnp.int32),
        pltpu.VMEM((CH,), _f32),
        pltpu.VMEM((16,), _f32),
        pltpu.VMEM((RPT,), _f32),
        pltpu.VMEM_SHARED((NP,), _f32),
        pltpu.VMEM_SHARED((NP,), _f32),
    ],
)


def _agg_body(h_hbm, src_hbm, dst_hbm, part_out,
              src_v, dst_v, rows_v, zb_v, sp_agg):
    c = lax.axis_index("c")
    s = lax.axis_index("s")
    wid = s * NC + c

    for r in range(16):
        for q in range(D // 16):
            zb_v[r, pl.ds(16 * q, 16)] = jnp.zeros((16,), _f32)

    def _z(k, _):
        pltpu.sync_copy(zb_v, sp_agg.at[pl.ds(s * RPT + 16 * k, 16)])
        return _
    lax.fori_loop(0, RPT // 16, _z, None)

    pltpu.sync_copy(src_hbm.at[wid], src_v)
    pltpu.sync_copy(dst_hbm.at[wid], dst_v)
    plsc.subcore_barrier()

    def _acc(j, _):
        pltpu.sync_copy(h_hbm.at[src_v.at[j]], rows_v)
        pltpu.sync_copy(rows_v, sp_agg.at[dst_v.at[j]], add=True)
        return _
    lax.fori_loop(0, NCHUNK, _acc, None)
    plsc.subcore_barrier()

    def _out(k, _):
        base = s * RPT + CH * k
        pltpu.sync_copy(sp_agg.at[pl.ds(base, CH)], rows_v)
        pltpu.sync_copy(rows_v, part_out.at[c, pl.ds(base, CH)])
        return _
    lax.fori_loop(0, RPT // CH, _out, None)


_sc_agg = pl.kernel(
    _agg_body,
    out_type=jax.ShapeDtypeStruct((NC, NP, D), _f32),
    mesh=_mesh,
    scratch_types=[
        pltpu.VMEM((NCHUNK, CH), jnp.int32),
        pltpu.VMEM((NCHUNK, CH), jnp.int32),
        pltpu.VMEM((CH, D), _f32),
        pltpu.VMEM((16, D), _f32),
        pltpu.VMEM_SHARED((NP, D), _f32),
    ],
)


def _scale_body(x_ref, odp_ref, idp_ref, hs_ref, ns_ref, nd_ref):
    od = odp_ref[0] + odp_ref[1]
    ind = idp_ref[0] + idp_ref[1]
    ns = lax.rsqrt(jnp.maximum(od, 1.0))
    nd = lax.rsqrt(jnp.maximum(ind, 1.0))
    hs_ref[...] = x_ref[...] * ns
    ns_ref[...] = ns
    nd_ref[...] = nd


def _tc_scale(x, odp, idp):
    return pl.pallas_call(
        _scale_body,
        grid=(NP // BR,),
        in_specs=[
            pl.BlockSpec((BR, D), lambda i: (i, 0)),
            pl.BlockSpec((NC, BR, 1), lambda i: (0, i, 0)),
            pl.BlockSpec((NC, BR, 1), lambda i: (0, i, 0)),
        ],
        out_specs=[
            pl.BlockSpec((BR, D), lambda i: (i, 0)),
            pl.BlockSpec((BR, 1), lambda i: (i, 0)),
            pl.BlockSpec((BR, 1), lambda i: (i, 0)),
        ],
        out_shape=[
            jax.ShapeDtypeStruct((NP, D), _f32),
            jax.ShapeDtypeStruct((NP, 1), _f32),
            jax.ShapeDtypeStruct((NP, 1), _f32),
        ],
    )(x, odp, idp)


def _layer_body(scale_out, part_ref, nd_ref, w_ref, b_ref, ns_ref, out_ref):
    agg = (part_ref[0] + part_ref[1]) * nd_ref[...]
    o = jnp.dot(agg, w_ref[...], preferred_element_type=_f32) + b_ref[...]
    o = jnp.where(o > 0, o, 0.01 * o)
    if scale_out:
        o = o * ns_ref[...]
    out_ref[...] = o


def _tc_layer(part, nd, w, b, ns, scale_out, n_out):
    return pl.pallas_call(
        functools.partial(_layer_body, scale_out),
        grid=(NP // BR,),
        in_specs=[
            pl.BlockSpec((NC, BR, D), lambda i: (0, i, 0)),
            pl.BlockSpec((BR, 1), lambda i: (i, 0)),
            pl.BlockSpec((D, D), lambda i: (0, 0)),
            pl.BlockSpec((1, D), lambda i: (0, 0)),
            pl.BlockSpec((BR, 1), lambda i: (i, 0)),
        ],
        out_specs=pl.BlockSpec((BR, D), lambda i: (i, 0)),
        out_shape=jax.ShapeDtypeStruct((n_out, D), _f32),
    )(part, nd, w, b, ns)


def kernel(x, edge_index, W0, b0, W1, b1):
    src = edge_index[0].reshape(NW, NCHUNK, CH)
    dst = edge_index[1].reshape(NW, NCHUNK, CH)

    odp, idp = _sc_deg(src, dst)
    odp = odp.reshape(NC, NP, 1)
    idp = idp.reshape(NC, NP, 1)

    h0s, ns, nd = _tc_scale(x, odp, idp)

    p0 = _sc_agg(h0s, src, dst)
    h1s = _tc_layer(p0, nd, W0, b0.reshape(1, D), ns, True, NP)

    p1 = _sc_agg(h1s, src, dst)
    out = _tc_layer(p1, nd, W1, b1.reshape(1, D), ns, False, N)
    return out
